# Initial kernel scaffold; baseline (speedup 1.0000x reference)
import jax
import jax.numpy as jnp
from jax.experimental import pallas as pl


def kernel(input_ids, attention_mask, emb, W1, b1, W2, b2, Wv, bv):
    raise NotImplementedError("write your pallas kernel here")


# speedup vs baseline: 4.9324x; 4.9324x over previous
import jax
import jax.numpy as jnp
from jax.experimental import pallas as pl
from jax.experimental.pallas import tpu as pltpu

_MASK_ID = 4
_KEEP = 0.1
_BLK = 512


def _classes(ids, msk):
    tidx = jnp.where(msk != 0, ids, 30)
    xidx = jnp.where(ids >= _MASK_ID, tidx, 31)
    return tidx, xidx


def _body(ids_ref, msk_ref, idsf_ref, mskf_ref, emb_ref, w1_ref, b1_ref,
          w2_ref, b2_ref, wv_ref, bv_ref, logits_ref, oids_ref,
          tT, tC, tP, nh, hr):
    i = pl.program_id(0)
    n_tok = (pl.num_programs(0) - 1) * _BLK

    @pl.when(i == 0)
    def _init():
        h0 = emb_ref[...]
        z = jnp.dot(h0, w1_ref[...],
                    preferred_element_type=jnp.float32) + b1_ref[...]
        ff = jnp.dot(jax.nn.gelu(z), w2_ref[...],
                     preferred_element_type=jnp.float32) + b2_ref[...]
        h = h0 + ff
        mu = jnp.mean(h, axis=-1, keepdims=True)
        var = jnp.mean((h - mu) * (h - mu), axis=-1, keepdims=True)
        hn = (h - mu) / jnp.sqrt(var + 1e-5)
        T = jnp.dot(hn, wv_ref[...],
                    preferred_element_type=jnp.float32) + bv_ref[...]
        tT[...] = T
        colv = jax.lax.broadcasted_iota(jnp.int32, (32, 128), 1)
        valid = colv < 30
        rmax = jnp.max(jnp.where(valid, T, -1e30), axis=-1, keepdims=True)
        e = jnp.where(valid, jnp.exp(T - rmax), 0.0)
        probs = e / jnp.sum(e, axis=-1, keepdims=True)
        cmax = jnp.max(probs, axis=-1, keepdims=True)
        pidx = jnp.min(jnp.where((probs == cmax) & valid, colv, 999),
                       axis=-1, keepdims=True)
        ri = jax.lax.broadcasted_iota(jnp.int32, (32, 32), 0)
        ci = jax.lax.broadcasted_iota(jnp.int32, (32, 32), 1)
        eye = (ri == ci).astype(jnp.float32)
        tdn = (((0,), (0,)), ((), ()))
        crow = jax.lax.dot_general(cmax, eye, tdn,
                                   precision=jax.lax.Precision.HIGHEST,
                                   preferred_element_type=jnp.float32)
        lane = jax.lax.broadcasted_iota(jnp.int32, (1, 32), 1)
        tC[...] = jnp.where(lane == 31, 0.0, crow)
        tP[...] = jax.lax.dot_general(pidx.astype(jnp.float32), eye, tdn,
                                      precision=jax.lax.Precision.HIGHEST,
                                      preferred_element_type=jnp.float32)
        idsf = idsf_ref[...]
        _, xidxf = _classes(idsf, mskf_ref[...])
        clsf = jax.lax.broadcasted_iota(jnp.int32, (idsf.shape[0], 32), 1)
        nh[...] = jnp.sum((xidxf == clsf).astype(jnp.float32),
                          axis=0, keepdims=True)
        hr[...] = jnp.zeros((1, 32), jnp.float32)
        logits_ref[...] = jnp.zeros_like(logits_ref)
        oids_ref[...] = jnp.zeros_like(oids_ref)

    @pl.when(i > 0)
    def _block():
        ids = ids_ref[...]
        tidx, xidx = _classes(ids, msk_ref[...])
        cls = jax.lax.broadcasted_iota(jnp.int32, (_BLK, 32), 1)
        Ox = (xidx == cls).astype(jnp.float32)
        Ot = (tidx == cls).astype(jnp.float32)
        lg = jnp.dot(Ot, tT[...], precision=jax.lax.Precision.HIGHEST,
                     preferred_element_type=jnp.float32)
        logits_ref[...] = lg[:, :30]
        Crow = tC[...]
        n_row = nh[...]
        conf = jnp.sum(Ox * Crow, axis=1, keepdims=True)
        gt = jnp.sum(jnp.where(Crow > conf, n_row, 0.0),
                     axis=1, keepdims=True)
        eqm = (Crow == conf).astype(jnp.float32)
        eprev = jnp.sum(eqm * hr[...], axis=1, keepdims=True)
        ri = jax.lax.broadcasted_iota(jnp.int32, (_BLK, _BLK), 0)
        ci = jax.lax.broadcasted_iota(jnp.int32, (_BLK, _BLK), 1)
        tri = (ci < ri).astype(jnp.float32)
        cs = jnp.dot(tri, Ox, preferred_element_type=jnp.float32)
        wblk = jnp.sum(cs * eqm, axis=1, keepdims=True)
        rank = gt + eprev + wblk
        meaningful = jnp.float32(n_tok) - n_row[0:1, 31:32]
        nk = jnp.floor(jnp.float32(_KEEP) * meaningful)
        pred = jnp.sum(Ot * tP[...], axis=1, keepdims=True)
        upd = (ids == _MASK_ID) & (rank < nk)
        oids_ref[...] = jnp.where(upd, pred.astype(jnp.int32), ids)
        hr[...] += jnp.sum(Ox, axis=0, keepdims=True)


def kernel(input_ids, attention_mask, emb, W1, b1, W2, b2, Wv, bv):
    B, S = input_ids.shape
    N = B * S
    V, D = emb.shape
    F = W1.shape[1]
    NB = N // _BLK

    ids2 = input_ids.reshape(N, 1)
    msk2 = attention_mask.reshape(N, 1).astype(jnp.int32)
    emb_p = jnp.zeros((32, D), emb.dtype).at[:V].set(emb)
    wv_p = jnp.zeros((D, 128), Wv.dtype).at[:, :V].set(Wv)
    bv_p = jnp.zeros((1, 128), bv.dtype).at[0, :V].set(bv)
    b1r = b1.reshape(1, F)
    b2r = b2.reshape(1, D)

    def blk_map(i):
        return (jnp.maximum(i - 1, 0), 0)

    logits2, oids2 = pl.pallas_call(
        _body,
        grid=(NB + 1,),
        in_specs=[
            pl.BlockSpec((_BLK, 1), blk_map),
            pl.BlockSpec((_BLK, 1), blk_map),
            pl.BlockSpec((N, 1), lambda i: (0, 0)),
            pl.BlockSpec((N, 1), lambda i: (0, 0)),
            pl.BlockSpec((32, D), lambda i: (0, 0)),
            pl.BlockSpec((D, F), lambda i: (0, 0)),
            pl.BlockSpec((1, F), lambda i: (0, 0)),
            pl.BlockSpec((F, D), lambda i: (0, 0)),
            pl.BlockSpec((1, D), lambda i: (0, 0)),
            pl.BlockSpec((D, 128), lambda i: (0, 0)),
            pl.BlockSpec((1, 128), lambda i: (0, 0)),
        ],
        out_specs=[
            pl.BlockSpec((_BLK, V), blk_map),
            pl.BlockSpec((_BLK, 1), blk_map),
        ],
        out_shape=[
            jax.ShapeDtypeStruct((N, V), jnp.float32),
            jax.ShapeDtypeStruct((N, 1), jnp.int32),
        ],
        scratch_shapes=[
            pltpu.VMEM((32, 128), jnp.float32),
            pltpu.VMEM((1, 32), jnp.float32),
            pltpu.VMEM((1, 32), jnp.float32),
            pltpu.VMEM((1, 32), jnp.float32),
            pltpu.VMEM((1, 32), jnp.float32),
        ],
        compiler_params=pltpu.CompilerParams(
            dimension_semantics=("arbitrary",)),
    )(ids2, msk2, ids2, msk2, emb_p, W1, b1r, W2, b2r, wv_p, bv_p)
    return logits2.reshape(B, S, V), oids2.reshape(B, S)


# speedup vs baseline: 5.2546x; 1.0653x over previous
import jax
import jax.numpy as jnp
from jax.experimental import pallas as pl
from jax.experimental.pallas import tpu as pltpu

_MASK_ID = 4
_KEEP = 0.1
_BLK = 1024


def _classes(ids, msk):
    tidx = jnp.where(msk != 0, ids, 30)
    xidx = jnp.where(ids >= _MASK_ID, tidx, 31)
    return tidx, xidx


def _body(ids_ref, msk_ref, idsf_ref, mskf_ref, emb_ref, w1_ref, b1_ref,
          w2_ref, b2_ref, wv_ref, bv_ref, logits_ref, oids_ref,
          tT, tC, tP, nh, hr):
    i = pl.program_id(0)
    n_tok = (pl.num_programs(0) - 1) * _BLK

    @pl.when(i == 0)
    def _init():
        h0 = emb_ref[...]
        z = jnp.dot(h0, w1_ref[...],
                    preferred_element_type=jnp.float32) + b1_ref[...]
        ff = jnp.dot(jax.nn.gelu(z), w2_ref[...],
                     preferred_element_type=jnp.float32) + b2_ref[...]
        h = h0 + ff
        mu = jnp.mean(h, axis=-1, keepdims=True)
        var = jnp.mean((h - mu) * (h - mu), axis=-1, keepdims=True)
        hn = (h - mu) / jnp.sqrt(var + 1e-5)
        T = jnp.dot(hn, wv_ref[...],
                    preferred_element_type=jnp.float32) + bv_ref[...]
        tT[...] = T
        colv = jax.lax.broadcasted_iota(jnp.int32, (32, 128), 1)
        valid = colv < 30
        rmax = jnp.max(jnp.where(valid, T, -1e30), axis=-1, keepdims=True)
        e = jnp.where(valid, jnp.exp(T - rmax), 0.0)
        probs = e / jnp.sum(e, axis=-1, keepdims=True)
        cmax = jnp.max(probs, axis=-1, keepdims=True)
        pidx = jnp.min(jnp.where((probs == cmax) & valid, colv, 999),
                       axis=-1, keepdims=True)
        ri = jax.lax.broadcasted_iota(jnp.int32, (32, 32), 0)
        ci = jax.lax.broadcasted_iota(jnp.int32, (32, 32), 1)
        eye = (ri == ci).astype(jnp.float32)
        tdn = (((0,), (0,)), ((), ()))
        crow = jax.lax.dot_general(cmax, eye, tdn,
                                   precision=jax.lax.Precision.HIGHEST,
                                   preferred_element_type=jnp.float32)
        lane = jax.lax.broadcasted_iota(jnp.int32, (1, 32), 1)
        tC[...] = jnp.where(lane == 31, 0.0, crow)
        tP[...] = jax.lax.dot_general(pidx.astype(jnp.float32), eye, tdn,
                                      precision=jax.lax.Precision.HIGHEST,
                                      preferred_element_type=jnp.float32)
        idsf = idsf_ref[...]
        _, xidxf = _classes(idsf, mskf_ref[...])
        clsf = jax.lax.broadcasted_iota(jnp.int32, (idsf.shape[0], 32), 1)
        nh[...] = jnp.sum((xidxf == clsf).astype(jnp.float32),
                          axis=0, keepdims=True)
        hr[...] = jnp.zeros((1, 32), jnp.float32)
        logits_ref[...] = jnp.zeros_like(logits_ref)
        oids_ref[...] = jnp.zeros_like(oids_ref)

    @pl.when(i > 0)
    def _block():
        ids = ids_ref[...]
        tidx, xidx = _classes(ids, msk_ref[...])
        cls = jax.lax.broadcasted_iota(jnp.int32, (_BLK, 32), 1)
        Ox = (xidx == cls).astype(jnp.float32)
        Ot = (tidx == cls).astype(jnp.float32)
        lg = jnp.dot(Ot, tT[...], precision=jax.lax.Precision.HIGHEST,
                     preferred_element_type=jnp.float32)
        logits_ref[...] = lg[:, :30]
        Crow = tC[...]
        n_row = nh[...]
        conf = jnp.sum(Ox * Crow, axis=1, keepdims=True)
        gt = jnp.sum(jnp.where(Crow > conf, n_row, 0.0),
                     axis=1, keepdims=True)
        eqm = (Crow == conf).astype(jnp.float32)
        eprev = jnp.sum(eqm * hr[...], axis=1, keepdims=True)
        ri = jax.lax.broadcasted_iota(jnp.int32, (_BLK, _BLK), 0)
        ci = jax.lax.broadcasted_iota(jnp.int32, (_BLK, _BLK), 1)
        tri = (ci < ri).astype(jnp.float32)
        cs = jnp.dot(tri, Ox, preferred_element_type=jnp.float32)
        wblk = jnp.sum(cs * eqm, axis=1, keepdims=True)
        rank = gt + eprev + wblk
        meaningful = jnp.float32(n_tok) - n_row[0:1, 31:32]
        nk = jnp.floor(jnp.float32(_KEEP) * meaningful)
        pred = jnp.sum(Ot * tP[...], axis=1, keepdims=True)
        upd = (ids == _MASK_ID) & (rank < nk)
        oids_ref[...] = jnp.where(upd, pred.astype(jnp.int32), ids)
        hr[...] += jnp.sum(Ox, axis=0, keepdims=True)


def kernel(input_ids, attention_mask, emb, W1, b1, W2, b2, Wv, bv):
    B, S = input_ids.shape
    N = B * S
    V, D = emb.shape
    F = W1.shape[1]
    NB = N // _BLK

    ids2 = input_ids.reshape(N, 1)
    msk2 = attention_mask.reshape(N, 1).astype(jnp.int32)
    emb_p = jnp.zeros((32, D), emb.dtype).at[:V].set(emb)
    wv_p = jnp.zeros((D, 128), Wv.dtype).at[:, :V].set(Wv)
    bv_p = jnp.zeros((1, 128), bv.dtype).at[0, :V].set(bv)
    b1r = b1.reshape(1, F)
    b2r = b2.reshape(1, D)

    def blk_map(i):
        return (jnp.maximum(i - 1, 0), 0)

    logits2, oids2 = pl.pallas_call(
        _body,
        grid=(NB + 1,),
        in_specs=[
            pl.BlockSpec((_BLK, 1), blk_map),
            pl.BlockSpec((_BLK, 1), blk_map),
            pl.BlockSpec((N, 1), lambda i: (0, 0)),
            pl.BlockSpec((N, 1), lambda i: (0, 0)),
            pl.BlockSpec((32, D), lambda i: (0, 0)),
            pl.BlockSpec((D, F), lambda i: (0, 0)),
            pl.BlockSpec((1, F), lambda i: (0, 0)),
            pl.BlockSpec((F, D), lambda i: (0, 0)),
            pl.BlockSpec((1, D), lambda i: (0, 0)),
            pl.BlockSpec((D, 128), lambda i: (0, 0)),
            pl.BlockSpec((1, 128), lambda i: (0, 0)),
        ],
        out_specs=[
            pl.BlockSpec((_BLK, V), blk_map),
            pl.BlockSpec((_BLK, 1), blk_map),
        ],
        out_shape=[
            jax.ShapeDtypeStruct((N, V), jnp.float32),
            jax.ShapeDtypeStruct((N, 1), jnp.int32),
        ],
        scratch_shapes=[
            pltpu.VMEM((32, 128), jnp.float32),
            pltpu.VMEM((1, 32), jnp.float32),
            pltpu.VMEM((1, 32), jnp.float32),
            pltpu.VMEM((1, 32), jnp.float32),
            pltpu.VMEM((1, 32), jnp.float32),
        ],
        compiler_params=pltpu.CompilerParams(
            dimension_semantics=("arbitrary",)),
    )(ids2, msk2, ids2, msk2, emb_p, W1, b1r, W2, b2r, wv_p, bv_p)
    return logits2.reshape(B, S, V), oids2.reshape(B, S)
